# baseline (speedup 1.0000x reference)
import jax
import jax.numpy as jnp
from jax.experimental import pallas as pl
from jax.experimental.pallas import tpu as pltpu


_TOPK = 10
_IDX_PAD = 128


def _policy_kernel(se_ref, wih_ref, b_ref, slen_ref, whh_ref, wout_ref,
                   bout_ref, probs_ref, h_ref, c_ref, idx_ref, h_scr, c_scr):
    t = pl.program_id(0)
    num_t = pl.num_programs(0)
    H = whh_ref.shape[0]
    Bb = h_scr.shape[0]
    C = wout_ref.shape[1]

    @pl.when(t == 0)
    def _init():
        h_scr[...] = jnp.zeros_like(h_scr)
        c_scr[...] = jnp.zeros_like(c_scr)

    h = h_scr[...]
    c = c_scr[...]

    gx = (jnp.dot(se_ref[0], wih_ref[...],
                  preferred_element_type=jnp.float32)
          + b_ref[...]).astype(jnp.bfloat16)
    gates = gx.astype(jnp.float32) + jnp.dot(
        h, whh_ref[...], preferred_element_type=jnp.float32)
    i_g = jax.nn.sigmoid(gates[:, 0 * H:1 * H])
    f_g = jax.nn.sigmoid(gates[:, 1 * H:2 * H])
    g_g = jnp.tanh(gates[:, 2 * H:3 * H])
    o_g = jax.nn.sigmoid(gates[:, 3 * H:4 * H])
    c_new = f_g * c + i_g * g_g
    h_new = (o_g * jnp.tanh(c_new)).astype(jnp.bfloat16)
    valid = t < slen_ref[...]
    h_scr[...] = jnp.where(valid, h_new, h)
    c_scr[...] = jnp.where(valid, c_new, c)

    @pl.when(t == num_t - 1)
    def _head():
        hf = h_scr[...].astype(jnp.float32)
        logits = jnp.dot(hf, wout_ref[...],
                         preferred_element_type=jnp.float32) + bout_ref[...]
        m = jnp.max(logits, axis=1, keepdims=True)
        e = jnp.exp(logits - m)
        probs_ref[...] = e / jnp.sum(e, axis=1, keepdims=True)
        h_ref[...] = hf
        c_ref[...] = c_scr[...]

        lane = jax.lax.broadcasted_iota(jnp.int32, (Bb, C), 1)
        vals = logits
        for k in range(_TOPK):
            mk = jnp.max(vals, axis=1, keepdims=True)
            idx_k = jnp.min(jnp.where(vals == mk, lane, C),
                            axis=1, keepdims=True)
            idx_ref[:, k:k + 1] = idx_k
            vals = jnp.where(lane == idx_k, -jnp.inf, vals)
        idx_ref[:, _TOPK:] = jnp.zeros((Bb, _IDX_PAD - _TOPK), jnp.int32)


def kernel(seq_idx, seq_len, embedding, w_ih, w_hh, b, w_out, b_out):
    T, B = seq_idx.shape
    E = embedding.shape[1]
    H = w_hh.shape[0]
    C = w_out.shape[1]

    seq_em = jnp.take(embedding, seq_idx, axis=0).astype(jnp.float32)
    slen = seq_len.astype(jnp.int32)[:, None]

    probs, h_last, c_last, idx_pad = pl.pallas_call(
        _policy_kernel,
        out_shape=(
            jax.ShapeDtypeStruct((B, C), jnp.float32),
            jax.ShapeDtypeStruct((B, H), jnp.float32),
            jax.ShapeDtypeStruct((B, H), jnp.float32),
            jax.ShapeDtypeStruct((B, _IDX_PAD), jnp.int32),
        ),
        grid=(T,),
        in_specs=[
            pl.BlockSpec((1, B, E), lambda t: (t, 0, 0)),
            pl.BlockSpec((E, 4 * H), lambda t: (0, 0)),
            pl.BlockSpec((1, 4 * H), lambda t: (0, 0)),
            pl.BlockSpec((B, 1), lambda t: (0, 0)),
            pl.BlockSpec((H, 4 * H), lambda t: (0, 0)),
            pl.BlockSpec((H, C), lambda t: (0, 0)),
            pl.BlockSpec((1, C), lambda t: (0, 0)),
        ],
        out_specs=(
            pl.BlockSpec((B, C), lambda t: (0, 0)),
            pl.BlockSpec((B, H), lambda t: (0, 0)),
            pl.BlockSpec((B, H), lambda t: (0, 0)),
            pl.BlockSpec((B, _IDX_PAD), lambda t: (0, 0)),
        ),
        scratch_shapes=[
            pltpu.VMEM((B, H), jnp.bfloat16),
            pltpu.VMEM((B, H), jnp.float32),
        ],
        compiler_params=pltpu.CompilerParams(
            dimension_semantics=("arbitrary",)),
    )(seq_em, w_ih.astype(jnp.float32), b.astype(jnp.float32), slen,
      w_hh.astype(jnp.bfloat16), w_out.astype(jnp.float32),
      b_out.astype(jnp.float32))

    indices = idx_pad[:, :_TOPK]
    return probs, indices, (h_last[None], c_last[None])


# speedup vs baseline: 1.1997x; 1.1997x over previous
import jax
import jax.numpy as jnp
from jax.experimental import pallas as pl
from jax.experimental.pallas import tpu as pltpu


_TOPK = 10


def _policy_kernel(gx_ref, slen_ref, whh_ref, wout_hbm, bout_ref,
                   probs_ref, h_ref, c_ref, idx_ref, wout_vmem, wout_sem):
    T, Bb, _G = gx_ref.shape
    H = whh_ref.shape[0]
    C = wout_vmem.shape[1]

    wout_cp = pltpu.make_async_copy(wout_hbm, wout_vmem, wout_sem)
    wout_cp.start()

    h0 = jnp.zeros((Bb, H), jnp.bfloat16)
    c0 = jnp.zeros((Bb, H), jnp.float32)
    slen = slen_ref[...]

    def step(t, carry):
        h, c = carry
        gates = gx_ref[t].astype(jnp.float32) + jnp.dot(
            h, whh_ref[...], preferred_element_type=jnp.float32)
        i_g = jax.nn.sigmoid(gates[:, 0 * H:1 * H])
        f_g = jax.nn.sigmoid(gates[:, 1 * H:2 * H])
        g_g = jnp.tanh(gates[:, 2 * H:3 * H])
        o_g = jax.nn.sigmoid(gates[:, 3 * H:4 * H])
        c_new = f_g * c + i_g * g_g
        h_new = (o_g * jnp.tanh(c_new)).astype(jnp.bfloat16)
        valid = t < slen
        return (jnp.where(valid, h_new, h), jnp.where(valid, c_new, c))

    h, c = jax.lax.fori_loop(0, T, step, (h0, c0), unroll=True)

    wout_cp.wait()
    hf = h.astype(jnp.float32)
    logits = jnp.dot(hf, wout_vmem[...],
                     preferred_element_type=jnp.float32) + bout_ref[...]
    m = jnp.max(logits, axis=1, keepdims=True)
    e = jnp.exp(logits - m)
    probs_ref[...] = e / jnp.sum(e, axis=1, keepdims=True)
    h_ref[0] = hf
    c_ref[0] = c

    lane = jax.lax.broadcasted_iota(jnp.int32, (Bb, C), 1)
    vals = logits
    mk = m
    for k in range(_TOPK):
        idx_k = jnp.min(jnp.where(vals == mk, lane, C), axis=1, keepdims=True)
        idx_ref[:, k:k + 1] = idx_k
        if k + 1 < _TOPK:
            vals = jnp.where(lane == idx_k, -jnp.inf, vals)
            mk = jnp.max(vals, axis=1, keepdims=True)


def kernel(seq_idx, seq_len, embedding, w_ih, w_hh, b, w_out, b_out):
    T, B = seq_idx.shape
    H = w_hh.shape[0]
    C = w_out.shape[1]
    Bb = B

    seq_em = jnp.take(embedding, seq_idx, axis=0).astype(jnp.float32)
    gates_x = (jnp.einsum("tbe,eg->tbg", seq_em, w_ih.astype(jnp.float32)) + b
               ).astype(jnp.bfloat16)
    slen = seq_len.astype(jnp.int32)[:, None]

    probs, h_last, c_last, indices = pl.pallas_call(
        _policy_kernel,
        out_shape=(
            jax.ShapeDtypeStruct((B, C), jnp.float32),
            jax.ShapeDtypeStruct((1, B, H), jnp.float32),
            jax.ShapeDtypeStruct((1, B, H), jnp.float32),
            jax.ShapeDtypeStruct((B, _TOPK), jnp.int32),
        ),
        grid=(1,),
        in_specs=[
            pl.BlockSpec((T, Bb, 4 * H), lambda i: (0, i, 0)),
            pl.BlockSpec((Bb, 1), lambda i: (i, 0)),
            pl.BlockSpec((H, 4 * H), lambda i: (0, 0)),
            pl.BlockSpec(memory_space=pl.ANY),
            pl.BlockSpec((1, C), lambda i: (0, 0)),
        ],
        out_specs=(
            pl.BlockSpec((Bb, C), lambda i: (i, 0)),
            pl.BlockSpec((1, Bb, H), lambda i: (0, i, 0)),
            pl.BlockSpec((1, Bb, H), lambda i: (0, i, 0)),
            pl.BlockSpec((Bb, _TOPK), lambda i: (i, 0)),
        ),
        scratch_shapes=[
            pltpu.VMEM((H, C), jnp.float32),
            pltpu.SemaphoreType.DMA,
        ],
        compiler_params=pltpu.CompilerParams(
            dimension_semantics=("arbitrary",)),
    )(gates_x, slen, w_hh.astype(jnp.bfloat16), w_out.astype(jnp.float32),
      b_out.astype(jnp.float32))

    return probs, indices, (h_last, c_last)
